# baseline (speedup 1.0000x reference)
import functools

import jax
import jax.numpy as jnp
from jax import lax
from jax.experimental import pallas as pl
from jax.experimental.pallas import tpu as pltpu
from jax.experimental.pallas import tpu_sc as plsc

N = 10000
E = 320000
D_IN = 128
H = 128
NH = 8
HD = H // NH
TD = 32
NL = 3
OUT = 2

NPAD = 10240
ROWS_BLK = 512

NC = 2
NS = 16
NW = NC * NS
CH = 128
NCHUNK = -(-E // (NW * CH))
PER_W = NCHUNK * CH
EP = NW * PER_W
ROWS_PER_TILE = NPAD // NS


def _proj_body(x_ref, w_ref, b_ref, o_ref):
    o_ref[...] = (
        jnp.dot(x_ref[...], w_ref[...], preferred_element_type=jnp.float32)
        + b_ref[...]
    )


def _proj(x, w, b):
    din = x.shape[1]
    dout = w.shape[1]
    return pl.pallas_call(
        _proj_body,
        grid=(NPAD // ROWS_BLK,),
        in_specs=[
            pl.BlockSpec((ROWS_BLK, din), lambda i: (i, 0)),
            pl.BlockSpec((din, dout), lambda i: (0, 0)),
            pl.BlockSpec((1, dout), lambda i: (0, 0)),
        ],
        out_specs=pl.BlockSpec((ROWS_BLK, dout), lambda i: (i, 0)),
        out_shape=jax.ShapeDtypeStruct((NPAD, dout), jnp.float32),
    )(x, w, b.reshape(1, dout))


_TF_BLK = 2048


_EM_BLK = 1024


def _edge_fused_body(qd_ref, ks_ref, vs_ref, msg_ref, exb_ref):
    p = qd_ref[...] * ks_ref[...]
    lane = lax.broadcasted_iota(jnp.int32, (H, NH), 0) // HD
    head = lax.broadcasted_iota(jnp.int32, (H, NH), 1)
    m_sum = (lane == head).astype(jnp.float32)
    s = jnp.dot(
        p, m_sum, preferred_element_type=jnp.float32) * (1.0 / (HD ** 0.5))
    ex = jnp.exp(s)
    lane2 = lax.broadcasted_iota(jnp.int32, (NH, H), 1) // HD
    head2 = lax.broadcasted_iota(jnp.int32, (NH, H), 0)
    m_exp = (lane2 == head2).astype(jnp.float32)
    exb = jnp.dot(ex, m_exp, preferred_element_type=jnp.float32)
    msg_ref[...] = exb * vs_ref[...]
    exb_ref[...] = exb


def _edge_fused(qd, ks, vs):
    return pl.pallas_call(
        _edge_fused_body,
        grid=(EP // _EM_BLK,),
        in_specs=[
            pl.BlockSpec((_EM_BLK, H), lambda i: (i, 0)),
            pl.BlockSpec((_EM_BLK, H), lambda i: (i, 0)),
            pl.BlockSpec((_EM_BLK, H), lambda i: (i, 0)),
        ],
        out_specs=[
            pl.BlockSpec((_EM_BLK, H), lambda i: (i, 0)),
            pl.BlockSpec((_EM_BLK, H), lambda i: (i, 0)),
        ],
        out_shape=[
            jax.ShapeDtypeStruct((EP, H), jnp.float32),
            jax.ShapeDtypeStruct((EP, H), jnp.float32),
        ],
    )(qd, ks, vs)


def _edge_score_body(qd_ref, ks_ref, o_ref):
    p = qd_ref[...] * ks_ref[...]
    lane = lax.broadcasted_iota(jnp.int32, (H, NH), 0) // HD
    head = lax.broadcasted_iota(jnp.int32, (H, NH), 1)
    m_sum = (lane == head).astype(jnp.float32)
    o_ref[...] = jnp.dot(
        p, m_sum, preferred_element_type=jnp.float32) * (1.0 / (HD ** 0.5))


def _edge_score(qd, ks):
    return pl.pallas_call(
        _edge_score_body,
        grid=(EP // _EM_BLK,),
        in_specs=[
            pl.BlockSpec((_EM_BLK, H), lambda i: (i, 0)),
            pl.BlockSpec((_EM_BLK, H), lambda i: (i, 0)),
        ],
        out_specs=pl.BlockSpec((_EM_BLK, NH), lambda i: (i, 0)),
        out_shape=jax.ShapeDtypeStruct((EP, NH), jnp.float32),
    )(qd, ks)


def _edge_apply_body(ex_ref, vs_ref, msg_ref, exb_ref):
    ex = ex_ref[...]
    lane2 = lax.broadcasted_iota(jnp.int32, (NH, H), 1) // HD
    head2 = lax.broadcasted_iota(jnp.int32, (NH, H), 0)
    m_exp = (lane2 == head2).astype(jnp.float32)
    exb = jnp.dot(ex, m_exp, preferred_element_type=jnp.float32)
    msg_ref[...] = exb * vs_ref[...]
    exb_ref[...] = exb


def _edge_apply(ex, vs):
    return pl.pallas_call(
        _edge_apply_body,
        grid=(EP // _EM_BLK,),
        in_specs=[
            pl.BlockSpec((_EM_BLK, NH), lambda i: (i, 0)),
            pl.BlockSpec((_EM_BLK, H), lambda i: (i, 0)),
        ],
        out_specs=[
            pl.BlockSpec((_EM_BLK, H), lambda i: (i, 0)),
            pl.BlockSpec((_EM_BLK, H), lambda i: (i, 0)),
        ],
        out_shape=[
            jax.ShapeDtypeStruct((EP, H), jnp.float32),
            jax.ShapeDtypeStruct((EP, H), jnp.float32),
        ],
    )(ex, vs)


_SC_SCATTER_CACHE = {}


def _sc_mesh():
    return plsc.VectorSubcoreMesh(
        core_axis_name="c", subcore_axis_name="s",
        num_cores=NC, num_subcores=NS)


def _make_sc_scatter():
    if "s" in _SC_SCATTER_CACHE:
        return _SC_SCATTER_CACHE["s"]

    @functools.partial(
        pl.kernel, mesh=_sc_mesh(),
        out_type=jax.ShapeDtypeStruct((NC * NPAD, H), jnp.float32),
        scratch_types=[
            pltpu.VMEM((CH, H), jnp.float32),
            pltpu.VMEM_SHARED((NPAD, H), jnp.float32),
            pltpu.VMEM((CH,), jnp.int32),
        ])
    def sc_scatter(msg_hbm, idx_hbm, zeros_hbm, out1, bufA, acc1, idx_v):
        cid = lax.axis_index("c")
        sid = lax.axis_index("s")
        wid = sid * NC + cid
        row0 = sid * ROWS_PER_TILE

        pltpu.sync_copy(zeros_hbm.at[pl.ds(0, CH)], bufA)

        def zbody(j, c):
            pltpu.sync_copy(bufA, acc1.at[pl.ds(row0 + j * CH, CH)])
            return c

        lax.fori_loop(0, ROWS_PER_TILE // CH, zbody, 0)
        plsc.subcore_barrier()

        def body(g, carry):
            base = wid * PER_W + g * CH
            pltpu.sync_copy(idx_hbm.at[pl.ds(base, CH)], idx_v)
            pltpu.sync_copy(msg_hbm.at[pl.ds(base, CH)], bufA)
            pltpu.sync_copy(bufA, acc1.at[idx_v], add=True)
            return carry

        lax.fori_loop(0, NCHUNK, body, 0)
        plsc.subcore_barrier()

        def fbody(j, c):
            pltpu.sync_copy(acc1.at[pl.ds(row0 + j * CH, CH)], bufA)
            pltpu.sync_copy(
                bufA, out1.at[pl.ds(cid * NPAD + row0 + j * CH, CH)])
            return c

        lax.fori_loop(0, ROWS_PER_TILE // CH, fbody, 0)

    def run(msg, idx):
        zeros = jnp.zeros((CH, H), jnp.float32)
        o1 = sc_scatter(msg, idx, zeros)
        return o1.reshape(NC, NPAD, H)

    _SC_SCATTER_CACHE["s"] = run
    return run


def _make_sc_gather3():
    if "g" in _SC_SCATTER_CACHE:
        return _SC_SCATTER_CACHE["g"]

    @functools.partial(
        pl.kernel, mesh=_sc_mesh(),
        out_type=(
            jax.ShapeDtypeStruct((EP, H), jnp.float32),
            jax.ShapeDtypeStruct((EP, H), jnp.float32),
            jax.ShapeDtypeStruct((EP, H), jnp.float32),
        ),
        scratch_types=[
            pltpu.VMEM((2, CH), jnp.int32),
            pltpu.VMEM((2, CH), jnp.int32),
            pltpu.VMEM((CH, H), jnp.float32),
            pltpu.VMEM((CH, H), jnp.float32),
            pltpu.VMEM((CH, H), jnp.float32),
            pltpu.VMEM((CH, H), jnp.float32),
            pltpu.VMEM((CH, H), jnp.float32),
            pltpu.VMEM((CH, H), jnp.float32),
            pltpu.SemaphoreType.DMA,
            pltpu.SemaphoreType.DMA,
            pltpu.SemaphoreType.DMA,
            pltpu.SemaphoreType.DMA,
        ])
    def sc_gather(q_hbm, k_hbm, v_hbm, dst_hbm, src_hbm,
                  qd_hbm, ks_hbm, vs_hbm,
                  idxd_v, idxs_v, bufQa, bufKa, bufVa, bufQb, bufKb, bufVb,
                  sga, sgb, swa, swb):
        cid = lax.axis_index("c")
        sid = lax.axis_index("s")
        wid = sid * NC + cid

        def pair(g2, carry):
            base_a = wid * PER_W + (2 * g2) * CH
            base_b = base_a + CH
            pltpu.sync_copy(dst_hbm.at[pl.ds(base_a, CH)], idxd_v.at[0])
            pltpu.sync_copy(src_hbm.at[pl.ds(base_a, CH)], idxs_v.at[0])
            pltpu.sync_copy(dst_hbm.at[pl.ds(base_b, CH)], idxd_v.at[1])
            pltpu.sync_copy(src_hbm.at[pl.ds(base_b, CH)], idxs_v.at[1])
            ga = [pltpu.async_copy(q_hbm.at[idxd_v.at[0]], bufQa, sga),
                  pltpu.async_copy(k_hbm.at[idxs_v.at[0]], bufKa, sga),
                  pltpu.async_copy(v_hbm.at[idxs_v.at[0]], bufVa, sga)]
            gb = [pltpu.async_copy(q_hbm.at[idxd_v.at[1]], bufQb, sgb),
                  pltpu.async_copy(k_hbm.at[idxs_v.at[1]], bufKb, sgb),
                  pltpu.async_copy(v_hbm.at[idxs_v.at[1]], bufVb, sgb)]
            for d in ga:
                d.wait()
            wa = [pltpu.async_copy(bufQa, qd_hbm.at[pl.ds(base_a, CH)], swa),
                  pltpu.async_copy(bufKa, ks_hbm.at[pl.ds(base_a, CH)], swa),
                  pltpu.async_copy(bufVa, vs_hbm.at[pl.ds(base_a, CH)], swa)]
            for d in gb:
                d.wait()
            wb = [pltpu.async_copy(bufQb, qd_hbm.at[pl.ds(base_b, CH)], swb),
                  pltpu.async_copy(bufKb, ks_hbm.at[pl.ds(base_b, CH)], swb),
                  pltpu.async_copy(bufVb, vs_hbm.at[pl.ds(base_b, CH)], swb)]
            for d in wa:
                d.wait()
            for d in wb:
                d.wait()
            return carry

        lax.fori_loop(0, NCHUNK // 2, pair, 0)

        base = wid * PER_W + (NCHUNK - 1) * CH
        pltpu.sync_copy(dst_hbm.at[pl.ds(base, CH)], idxd_v.at[0])
        pltpu.sync_copy(src_hbm.at[pl.ds(base, CH)], idxs_v.at[0])
        gt = [pltpu.async_copy(q_hbm.at[idxd_v.at[0]], bufQa, sga),
              pltpu.async_copy(k_hbm.at[idxs_v.at[0]], bufKa, sga),
              pltpu.async_copy(v_hbm.at[idxs_v.at[0]], bufVa, sga)]
        for d in gt:
            d.wait()
        pltpu.sync_copy(bufQa, qd_hbm.at[pl.ds(base, CH)])
        pltpu.sync_copy(bufKa, ks_hbm.at[pl.ds(base, CH)])
        pltpu.sync_copy(bufVa, vs_hbm.at[pl.ds(base, CH)])

    _SC_SCATTER_CACHE["g"] = sc_gather
    return sc_gather


def _norm_proj_body(p0_ref, p1_ref, s0_ref, s1_ref, w_ref, b_ref, o_ref):
    uagg = p0_ref[...] + p1_ref[...]
    ssb = s0_ref[...] + s1_ref[...]
    ssb = jnp.where(ssb == 0.0, 1.0, ssb)
    agg = uagg / ssb
    o_ref[...] = jnp.dot(
        agg, w_ref[...], preferred_element_type=jnp.float32) + b_ref[...]


def _norm_proj(p128, s128, w, b):
    return pl.pallas_call(
        _norm_proj_body,
        grid=(NPAD // ROWS_BLK,),
        in_specs=[
            pl.BlockSpec((ROWS_BLK, H), lambda i: (i, 0)),
            pl.BlockSpec((ROWS_BLK, H), lambda i: (i, 0)),
            pl.BlockSpec((ROWS_BLK, H), lambda i: (i, 0)),
            pl.BlockSpec((ROWS_BLK, H), lambda i: (i, 0)),
            pl.BlockSpec((H, H), lambda i: (0, 0)),
            pl.BlockSpec((1, H), lambda i: (0, 0)),
        ],
        out_specs=pl.BlockSpec((ROWS_BLK, H), lambda i: (i, 0)),
        out_shape=jax.ShapeDtypeStruct((NPAD, H), jnp.float32),
    )(p128[0], p128[1], s128[0], s128[1], w, b.reshape(1, H))


def kernel(x, edge_index, batch, timestamps, W_in, b_in, time_freq,
           Wq, bq, Wk, bk, Wv, bv, Wo, bo, W1, b1, W2, b2):
    src = edge_index[0]
    dst = edge_index[1]
    pad_idx = jnp.full((EP - E,), NPAD - 1, jnp.int32)
    src_p = jnp.concatenate([src, pad_idx])
    dst_p = jnp.concatenate([dst, pad_idx])

    t_norm = (timestamps - jnp.min(timestamps)) / (
        jnp.max(timestamps) - jnp.min(timestamps) + 1e-08)
    om = t_norm[:, None] * time_freq[None, :]
    tf_e = jnp.concatenate(
        [jnp.cos(om), jnp.sin(om), jnp.ones((E, 1), jnp.float32),
         jnp.zeros((E, H - TD - 1), jnp.float32)], axis=1)
    msg_tf = jnp.concatenate(
        [tf_e, jnp.zeros((EP - E, H), jnp.float32)], axis=0)
    tf_part = _make_sc_scatter()(msg_tf, dst_p)
    tot = tf_part[0] + tf_part[1]
    node_tf_p = tot[:, :TD] / (tot[:, TD:TD + 1] + 1e-08)

    xp = jnp.zeros((NPAD, D_IN), jnp.float32).at[:N].set(x)
    h = _proj(xp, W_in, b_in)

    for l in range(NL):
        xt = jnp.concatenate([h, node_tf_p], axis=-1)
        Q = _proj(xt, Wq[l], bq[l])
        K = _proj(xt, Wk[l], bk[l])
        V = _proj(h, Wv[l], bv[l])
        Qd, Ks, Vs = _make_sc_gather3()(Q, K, V, dst_p, src_p)
        msg, exb = _edge_fused(Qd, Ks, Vs)
        p128 = _make_sc_scatter()(msg, dst_p)
        s128 = _make_sc_scatter()(exb, dst_p)
        out = _norm_proj((p128[0], p128[1]), (s128[0], s128[1]), Wo[l], bo[l])
        h = jax.nn.elu(out)

    hN = h[:N]
    g = jnp.mean(hN, axis=0, keepdims=True)
    logits = jax.nn.relu(g @ W1 + b1) @ W2 + b2
    return logits


# speedup vs baseline: 1.0947x; 1.0947x over previous
import functools

import jax
import jax.numpy as jnp
from jax import lax
from jax.experimental import pallas as pl
from jax.experimental.pallas import tpu as pltpu
from jax.experimental.pallas import tpu_sc as plsc

N = 10000
E = 320000
D_IN = 128
H = 128
NH = 8
HD = H // NH
TD = 32
NL = 3
OUT = 2

NPAD = 10240
ROWS_BLK = 512

NC = 2
NS = 16
NW = NC * NS
CH = 128
NCHUNK = -(-E // (NW * CH))
PER_W = NCHUNK * CH
EP = NW * PER_W
ROWS_PER_TILE = NPAD // NS


def _proj_body(x_ref, w_ref, b_ref, o_ref):
    o_ref[...] = (
        jnp.dot(x_ref[...], w_ref[...], preferred_element_type=jnp.float32)
        + b_ref[...]
    )


def _proj(x, w, b):
    din = x.shape[1]
    dout = w.shape[1]
    return pl.pallas_call(
        _proj_body,
        grid=(NPAD // ROWS_BLK,),
        in_specs=[
            pl.BlockSpec((ROWS_BLK, din), lambda i: (i, 0)),
            pl.BlockSpec((din, dout), lambda i: (0, 0)),
            pl.BlockSpec((1, dout), lambda i: (0, 0)),
        ],
        out_specs=pl.BlockSpec((ROWS_BLK, dout), lambda i: (i, 0)),
        out_shape=jax.ShapeDtypeStruct((NPAD, dout), jnp.float32),
    )(x, w, b.reshape(1, dout))


_TF_BLK = 2048


_EM_BLK = 1024


def _edge_fused_body(qd_ref, ks_ref, vs_ref, msg_ref, exb_ref):
    p = qd_ref[...] * ks_ref[...]
    lane = lax.broadcasted_iota(jnp.int32, (H, NH), 0) // HD
    head = lax.broadcasted_iota(jnp.int32, (H, NH), 1)
    m_sum = (lane == head).astype(jnp.float32)
    s = jnp.dot(
        p, m_sum, preferred_element_type=jnp.float32) * (1.0 / (HD ** 0.5))
    ex = jnp.exp(s)
    lane2 = lax.broadcasted_iota(jnp.int32, (NH, H), 1) // HD
    head2 = lax.broadcasted_iota(jnp.int32, (NH, H), 0)
    m_exp = (lane2 == head2).astype(jnp.float32)
    exb = jnp.dot(ex, m_exp, preferred_element_type=jnp.float32)
    msg_ref[...] = exb * vs_ref[...]
    exb_ref[...] = exb


def _edge_fused(qd, ks, vs):
    return pl.pallas_call(
        _edge_fused_body,
        grid=(EP // _EM_BLK,),
        in_specs=[
            pl.BlockSpec((_EM_BLK, H), lambda i: (i, 0)),
            pl.BlockSpec((_EM_BLK, H), lambda i: (i, 0)),
            pl.BlockSpec((_EM_BLK, H), lambda i: (i, 0)),
        ],
        out_specs=[
            pl.BlockSpec((_EM_BLK, H), lambda i: (i, 0)),
            pl.BlockSpec((_EM_BLK, H), lambda i: (i, 0)),
        ],
        out_shape=[
            jax.ShapeDtypeStruct((EP, H), jnp.float32),
            jax.ShapeDtypeStruct((EP, H), jnp.float32),
        ],
    )(qd, ks, vs)


def _edge_score_body(qd_ref, ks_ref, o_ref):
    p = qd_ref[...] * ks_ref[...]
    lane = lax.broadcasted_iota(jnp.int32, (H, NH), 0) // HD
    head = lax.broadcasted_iota(jnp.int32, (H, NH), 1)
    m_sum = (lane == head).astype(jnp.float32)
    o_ref[...] = jnp.dot(
        p, m_sum, preferred_element_type=jnp.float32) * (1.0 / (HD ** 0.5))


def _edge_score(qd, ks):
    return pl.pallas_call(
        _edge_score_body,
        grid=(EP // _EM_BLK,),
        in_specs=[
            pl.BlockSpec((_EM_BLK, H), lambda i: (i, 0)),
            pl.BlockSpec((_EM_BLK, H), lambda i: (i, 0)),
        ],
        out_specs=pl.BlockSpec((_EM_BLK, NH), lambda i: (i, 0)),
        out_shape=jax.ShapeDtypeStruct((EP, NH), jnp.float32),
    )(qd, ks)


def _edge_apply_body(ex_ref, vs_ref, msg_ref, exb_ref):
    ex = ex_ref[...]
    lane2 = lax.broadcasted_iota(jnp.int32, (NH, H), 1) // HD
    head2 = lax.broadcasted_iota(jnp.int32, (NH, H), 0)
    m_exp = (lane2 == head2).astype(jnp.float32)
    exb = jnp.dot(ex, m_exp, preferred_element_type=jnp.float32)
    msg_ref[...] = exb * vs_ref[...]
    exb_ref[...] = exb


def _edge_apply(ex, vs):
    return pl.pallas_call(
        _edge_apply_body,
        grid=(EP // _EM_BLK,),
        in_specs=[
            pl.BlockSpec((_EM_BLK, NH), lambda i: (i, 0)),
            pl.BlockSpec((_EM_BLK, H), lambda i: (i, 0)),
        ],
        out_specs=[
            pl.BlockSpec((_EM_BLK, H), lambda i: (i, 0)),
            pl.BlockSpec((_EM_BLK, H), lambda i: (i, 0)),
        ],
        out_shape=[
            jax.ShapeDtypeStruct((EP, H), jnp.float32),
            jax.ShapeDtypeStruct((EP, H), jnp.float32),
        ],
    )(ex, vs)


_SC_SCATTER_CACHE = {}


def _sc_mesh():
    return plsc.VectorSubcoreMesh(
        core_axis_name="c", subcore_axis_name="s",
        num_cores=NC, num_subcores=NS)


def _make_sc_scatter():
    if "s" in _SC_SCATTER_CACHE:
        return _SC_SCATTER_CACHE["s"]

    @functools.partial(
        pl.kernel, mesh=_sc_mesh(),
        out_type=jax.ShapeDtypeStruct((NC * NPAD, H), jnp.float32),
        scratch_types=[
            pltpu.VMEM((CH, H), jnp.float32),
            pltpu.VMEM_SHARED((NPAD, H), jnp.float32),
            pltpu.VMEM((CH,), jnp.int32),
            pltpu.VMEM((CH, H), jnp.float32),
            pltpu.VMEM((CH,), jnp.int32),
            pltpu.SemaphoreType.DMA,
            pltpu.SemaphoreType.DMA,
        ])
    def sc_scatter(msg_hbm, idx_hbm, zeros_hbm, out1, bufA, acc1, idx_v,
                   bufB, idx_w, sma, smb):
        cid = lax.axis_index("c")
        sid = lax.axis_index("s")
        wid = sid * NC + cid
        row0 = sid * ROWS_PER_TILE

        pltpu.sync_copy(zeros_hbm.at[pl.ds(0, CH)], bufA)

        def zbody(j, c):
            pltpu.sync_copy(bufA, acc1.at[pl.ds(row0 + j * CH, CH)])
            return c

        lax.fori_loop(0, ROWS_PER_TILE // CH, zbody, 0)
        plsc.subcore_barrier()

        def pair(g2, carry):
            base_a = wid * PER_W + (2 * g2) * CH
            base_b = base_a + CH
            pltpu.sync_copy(idx_hbm.at[pl.ds(base_a, CH)], idx_v)
            pltpu.sync_copy(idx_hbm.at[pl.ds(base_b, CH)], idx_w)
            ma = pltpu.async_copy(msg_hbm.at[pl.ds(base_a, CH)], bufA, sma)
            mb = pltpu.async_copy(msg_hbm.at[pl.ds(base_b, CH)], bufB, smb)
            ma.wait()
            pltpu.sync_copy(bufA, acc1.at[idx_v], add=True)
            mb.wait()
            pltpu.sync_copy(bufB, acc1.at[idx_w], add=True)
            return carry

        lax.fori_loop(0, NCHUNK // 2, pair, 0)
        base = wid * PER_W + (NCHUNK - 1) * CH
        pltpu.sync_copy(idx_hbm.at[pl.ds(base, CH)], idx_v)
        pltpu.sync_copy(msg_hbm.at[pl.ds(base, CH)], bufA)
        pltpu.sync_copy(bufA, acc1.at[idx_v], add=True)
        plsc.subcore_barrier()

        def fbody(j, c):
            pltpu.sync_copy(acc1.at[pl.ds(row0 + j * CH, CH)], bufA)
            pltpu.sync_copy(
                bufA, out1.at[pl.ds(cid * NPAD + row0 + j * CH, CH)])
            return c

        lax.fori_loop(0, ROWS_PER_TILE // CH, fbody, 0)

    def run(msg, idx):
        zeros = jnp.zeros((CH, H), jnp.float32)
        o1 = sc_scatter(msg, idx, zeros)
        return o1.reshape(NC, NPAD, H)

    _SC_SCATTER_CACHE["s"] = run
    return run


def _make_sc_gather3():
    if "g" in _SC_SCATTER_CACHE:
        return _SC_SCATTER_CACHE["g"]

    @functools.partial(
        pl.kernel, mesh=_sc_mesh(),
        out_type=(
            jax.ShapeDtypeStruct((EP, H), jnp.float32),
            jax.ShapeDtypeStruct((EP, H), jnp.float32),
            jax.ShapeDtypeStruct((EP, H), jnp.float32),
        ),
        scratch_types=[
            pltpu.VMEM((CH,), jnp.int32),
            pltpu.VMEM((CH,), jnp.int32),
            pltpu.VMEM((CH, H), jnp.float32),
            pltpu.VMEM((CH, H), jnp.float32),
            pltpu.VMEM((CH, H), jnp.float32),
            pltpu.SemaphoreType.DMA,
            pltpu.SemaphoreType.DMA,
            pltpu.SemaphoreType.DMA,
        ])
    def sc_gather(q_hbm, k_hbm, v_hbm, dst_hbm, src_hbm,
                  qd_hbm, ks_hbm, vs_hbm,
                  idxd_v, idxs_v, bufQ, bufK, bufV, sem1, sem2, sem3):
        cid = lax.axis_index("c")
        sid = lax.axis_index("s")
        wid = sid * NC + cid

        def body(g, carry):
            base = wid * PER_W + g * CH
            pltpu.sync_copy(dst_hbm.at[pl.ds(base, CH)], idxd_v)
            pltpu.sync_copy(src_hbm.at[pl.ds(base, CH)], idxs_v)
            d1 = pltpu.async_copy(q_hbm.at[idxd_v], bufQ, sem1)
            d2 = pltpu.async_copy(k_hbm.at[idxs_v], bufK, sem2)
            d3 = pltpu.async_copy(v_hbm.at[idxs_v], bufV, sem3)
            d1.wait()
            d2.wait()
            d3.wait()
            pltpu.sync_copy(bufQ, qd_hbm.at[pl.ds(base, CH)])
            pltpu.sync_copy(bufK, ks_hbm.at[pl.ds(base, CH)])
            pltpu.sync_copy(bufV, vs_hbm.at[pl.ds(base, CH)])
            return carry

        lax.fori_loop(0, NCHUNK, body, 0)

    _SC_SCATTER_CACHE["g"] = sc_gather
    return sc_gather


def _norm_proj_body(p0_ref, p1_ref, s0_ref, s1_ref, w_ref, b_ref, o_ref):
    uagg = p0_ref[...] + p1_ref[...]
    ssb = s0_ref[...] + s1_ref[...]
    ssb = jnp.where(ssb == 0.0, 1.0, ssb)
    agg = uagg / ssb
    o_ref[...] = jnp.dot(
        agg, w_ref[...], preferred_element_type=jnp.float32) + b_ref[...]


def _norm_proj(p128, s128, w, b):
    return pl.pallas_call(
        _norm_proj_body,
        grid=(NPAD // ROWS_BLK,),
        in_specs=[
            pl.BlockSpec((ROWS_BLK, H), lambda i: (i, 0)),
            pl.BlockSpec((ROWS_BLK, H), lambda i: (i, 0)),
            pl.BlockSpec((ROWS_BLK, H), lambda i: (i, 0)),
            pl.BlockSpec((ROWS_BLK, H), lambda i: (i, 0)),
            pl.BlockSpec((H, H), lambda i: (0, 0)),
            pl.BlockSpec((1, H), lambda i: (0, 0)),
        ],
        out_specs=pl.BlockSpec((ROWS_BLK, H), lambda i: (i, 0)),
        out_shape=jax.ShapeDtypeStruct((NPAD, H), jnp.float32),
    )(p128[0], p128[1], s128[0], s128[1], w, b.reshape(1, H))


def kernel(x, edge_index, batch, timestamps, W_in, b_in, time_freq,
           Wq, bq, Wk, bk, Wv, bv, Wo, bo, W1, b1, W2, b2):
    src = edge_index[0]
    dst = edge_index[1]
    pad_idx = jnp.full((EP - E,), NPAD - 1, jnp.int32)
    src_p = jnp.concatenate([src, pad_idx])
    dst_p = jnp.concatenate([dst, pad_idx])

    t_norm = (timestamps - jnp.min(timestamps)) / (
        jnp.max(timestamps) - jnp.min(timestamps) + 1e-08)
    om = t_norm[:, None] * time_freq[None, :]
    tf_e = jnp.concatenate(
        [jnp.cos(om), jnp.sin(om), jnp.ones((E, 1), jnp.float32),
         jnp.zeros((E, H - TD - 1), jnp.float32)], axis=1)
    msg_tf = jnp.concatenate(
        [tf_e, jnp.zeros((EP - E, H), jnp.float32)], axis=0)
    tf_part = _make_sc_scatter()(msg_tf, dst_p)
    tot = tf_part[0] + tf_part[1]
    node_tf_p = tot[:, :TD] / (tot[:, TD:TD + 1] + 1e-08)

    xp = jnp.zeros((NPAD, D_IN), jnp.float32).at[:N].set(x)
    h = _proj(xp, W_in, b_in)

    for l in range(NL):
        xt = jnp.concatenate([h, node_tf_p], axis=-1)
        Q = _proj(xt, Wq[l], bq[l])
        K = _proj(xt, Wk[l], bk[l])
        V = _proj(h, Wv[l], bv[l])
        Qd, Ks, Vs = _make_sc_gather3()(Q, K, V, dst_p, src_p)
        msg, exb = _edge_fused(Qd, Ks, Vs)
        p128 = _make_sc_scatter()(msg, dst_p)
        s128 = _make_sc_scatter()(exb, dst_p)
        out = _norm_proj((p128[0], p128[1]), (s128[0], s128[1]), Wo[l], bo[l])
        h = jax.nn.elu(out)

    hN = h[:N]
    g = jnp.mean(hN, axis=0, keepdims=True)
    logits = jax.nn.relu(g @ W1 + b1) @ W2 + b2
    return logits


# speedup vs baseline: 1.1111x; 1.0150x over previous
import functools

import jax
import jax.numpy as jnp
from jax import lax
from jax.experimental import pallas as pl
from jax.experimental.pallas import tpu as pltpu
from jax.experimental.pallas import tpu_sc as plsc

N = 10000
E = 320000
D_IN = 128
H = 128
NH = 8
HD = H // NH
TD = 32
NL = 3
OUT = 2

NPAD = 10240
ROWS_BLK = 512

NC = 2
NS = 16
NW = NC * NS
CH = 128
NCHUNK = -(-E // (NW * CH))
PER_W = NCHUNK * CH
EP = NW * PER_W
ROWS_PER_TILE = NPAD // NS


def _proj_body(x_ref, w_ref, b_ref, o_ref):
    o_ref[...] = (
        jnp.dot(x_ref[...], w_ref[...], preferred_element_type=jnp.float32)
        + b_ref[...]
    )


def _proj(x, w, b):
    din = x.shape[1]
    dout = w.shape[1]
    return pl.pallas_call(
        _proj_body,
        grid=(NPAD // ROWS_BLK,),
        in_specs=[
            pl.BlockSpec((ROWS_BLK, din), lambda i: (i, 0)),
            pl.BlockSpec((din, dout), lambda i: (0, 0)),
            pl.BlockSpec((1, dout), lambda i: (0, 0)),
        ],
        out_specs=pl.BlockSpec((ROWS_BLK, dout), lambda i: (i, 0)),
        out_shape=jax.ShapeDtypeStruct((NPAD, dout), jnp.float32),
    )(x, w, b.reshape(1, dout))


_TF_BLK = 2048


_EM_BLK = 1024


def _edge_fused_body(qd_ref, ks_ref, vs_ref, msg_ref, exb_ref):
    p = qd_ref[...] * ks_ref[...]
    lane = lax.broadcasted_iota(jnp.int32, (H, NH), 0) // HD
    head = lax.broadcasted_iota(jnp.int32, (H, NH), 1)
    m_sum = (lane == head).astype(jnp.float32)
    s = jnp.dot(
        p, m_sum, preferred_element_type=jnp.float32) * (1.0 / (HD ** 0.5))
    ex = jnp.exp(s)
    lane2 = lax.broadcasted_iota(jnp.int32, (NH, H), 1) // HD
    head2 = lax.broadcasted_iota(jnp.int32, (NH, H), 0)
    m_exp = (lane2 == head2).astype(jnp.float32)
    exb = jnp.dot(ex, m_exp, preferred_element_type=jnp.float32)
    msg_ref[...] = exb * vs_ref[...]
    exb_ref[...] = exb


def _edge_fused(qd, ks, vs):
    return pl.pallas_call(
        _edge_fused_body,
        grid=(EP // _EM_BLK,),
        in_specs=[
            pl.BlockSpec((_EM_BLK, H), lambda i: (i, 0)),
            pl.BlockSpec((_EM_BLK, H), lambda i: (i, 0)),
            pl.BlockSpec((_EM_BLK, H), lambda i: (i, 0)),
        ],
        out_specs=[
            pl.BlockSpec((_EM_BLK, H), lambda i: (i, 0)),
            pl.BlockSpec((_EM_BLK, H), lambda i: (i, 0)),
        ],
        out_shape=[
            jax.ShapeDtypeStruct((EP, H), jnp.float32),
            jax.ShapeDtypeStruct((EP, H), jnp.float32),
        ],
    )(qd, ks, vs)


def _edge_score_body(qd_ref, ks_ref, o_ref):
    p = qd_ref[...] * ks_ref[...]
    lane = lax.broadcasted_iota(jnp.int32, (H, NH), 0) // HD
    head = lax.broadcasted_iota(jnp.int32, (H, NH), 1)
    m_sum = (lane == head).astype(jnp.float32)
    o_ref[...] = jnp.dot(
        p, m_sum, preferred_element_type=jnp.float32) * (1.0 / (HD ** 0.5))


def _edge_score(qd, ks):
    return pl.pallas_call(
        _edge_score_body,
        grid=(EP // _EM_BLK,),
        in_specs=[
            pl.BlockSpec((_EM_BLK, H), lambda i: (i, 0)),
            pl.BlockSpec((_EM_BLK, H), lambda i: (i, 0)),
        ],
        out_specs=pl.BlockSpec((_EM_BLK, NH), lambda i: (i, 0)),
        out_shape=jax.ShapeDtypeStruct((EP, NH), jnp.float32),
    )(qd, ks)


def _edge_apply_body(ex_ref, vs_ref, msg_ref, exb_ref):
    ex = ex_ref[...]
    lane2 = lax.broadcasted_iota(jnp.int32, (NH, H), 1) // HD
    head2 = lax.broadcasted_iota(jnp.int32, (NH, H), 0)
    m_exp = (lane2 == head2).astype(jnp.float32)
    exb = jnp.dot(ex, m_exp, preferred_element_type=jnp.float32)
    msg_ref[...] = exb * vs_ref[...]
    exb_ref[...] = exb


def _edge_apply(ex, vs):
    return pl.pallas_call(
        _edge_apply_body,
        grid=(EP // _EM_BLK,),
        in_specs=[
            pl.BlockSpec((_EM_BLK, NH), lambda i: (i, 0)),
            pl.BlockSpec((_EM_BLK, H), lambda i: (i, 0)),
        ],
        out_specs=[
            pl.BlockSpec((_EM_BLK, H), lambda i: (i, 0)),
            pl.BlockSpec((_EM_BLK, H), lambda i: (i, 0)),
        ],
        out_shape=[
            jax.ShapeDtypeStruct((EP, H), jnp.float32),
            jax.ShapeDtypeStruct((EP, H), jnp.float32),
        ],
    )(ex, vs)


_SC_SCATTER_CACHE = {}


def _sc_mesh():
    return plsc.VectorSubcoreMesh(
        core_axis_name="c", subcore_axis_name="s",
        num_cores=NC, num_subcores=NS)


def _make_sc_scatter():
    if "s" in _SC_SCATTER_CACHE:
        return _SC_SCATTER_CACHE["s"]

    @functools.partial(
        pl.kernel, mesh=_sc_mesh(),
        out_type=jax.ShapeDtypeStruct((NC * NPAD, H), jnp.float32),
        scratch_types=[
            pltpu.VMEM((CH, H), jnp.float32),
            pltpu.VMEM_SHARED((NPAD, H), jnp.float32),
            pltpu.VMEM((CH,), jnp.int32),
            pltpu.VMEM((CH, H), jnp.float32),
            pltpu.VMEM((CH,), jnp.int32),
            pltpu.SemaphoreType.DMA,
            pltpu.SemaphoreType.DMA,
        ])
    def sc_scatter(msg_hbm, idx_hbm, zeros_hbm, out1, bufA, acc1, idx_v,
                   bufB, idx_w, sma, smb):
        cid = lax.axis_index("c")
        sid = lax.axis_index("s")
        wid = sid * NC + cid
        row0 = sid * ROWS_PER_TILE

        pltpu.sync_copy(zeros_hbm.at[pl.ds(0, CH)], bufA)

        def zbody(j, c):
            pltpu.sync_copy(bufA, acc1.at[pl.ds(row0 + j * CH, CH)])
            return c

        lax.fori_loop(0, ROWS_PER_TILE // CH, zbody, 0)
        plsc.subcore_barrier()

        def pair(g2, carry):
            base_a = wid * PER_W + (2 * g2) * CH
            base_b = base_a + CH
            pltpu.sync_copy(idx_hbm.at[pl.ds(base_a, CH)], idx_v)
            pltpu.sync_copy(idx_hbm.at[pl.ds(base_b, CH)], idx_w)
            ma = pltpu.async_copy(msg_hbm.at[pl.ds(base_a, CH)], bufA, sma)
            mb = pltpu.async_copy(msg_hbm.at[pl.ds(base_b, CH)], bufB, smb)
            ma.wait()
            pltpu.sync_copy(bufA, acc1.at[idx_v], add=True)
            mb.wait()
            pltpu.sync_copy(bufB, acc1.at[idx_w], add=True)
            return carry

        lax.fori_loop(0, NCHUNK // 2, pair, 0)
        base = wid * PER_W + (NCHUNK - 1) * CH
        pltpu.sync_copy(idx_hbm.at[pl.ds(base, CH)], idx_v)
        pltpu.sync_copy(msg_hbm.at[pl.ds(base, CH)], bufA)
        pltpu.sync_copy(bufA, acc1.at[idx_v], add=True)
        plsc.subcore_barrier()

        def fbody(j, c):
            pltpu.sync_copy(acc1.at[pl.ds(row0 + j * CH, CH)], bufA)
            pltpu.sync_copy(
                bufA, out1.at[pl.ds(cid * NPAD + row0 + j * CH, CH)])
            return c

        lax.fori_loop(0, ROWS_PER_TILE // CH, fbody, 0)

    def run(msg, idx):
        zeros = jnp.zeros((CH, H), jnp.float32)
        o1 = sc_scatter(msg, idx, zeros)
        return o1.reshape(NC, NPAD, H)

    _SC_SCATTER_CACHE["s"] = run
    return run


def _make_sc_gather3():
    if "g" in _SC_SCATTER_CACHE:
        return _SC_SCATTER_CACHE["g"]

    @functools.partial(
        pl.kernel, mesh=_sc_mesh(),
        out_type=(
            jax.ShapeDtypeStruct((EP, H), jnp.float32),
            jax.ShapeDtypeStruct((EP, H), jnp.float32),
            jax.ShapeDtypeStruct((EP, H), jnp.float32),
        ),
        scratch_types=[
            pltpu.VMEM((CH,), jnp.int32),
            pltpu.VMEM((CH,), jnp.int32),
            pltpu.VMEM((CH, H), jnp.float32),
            pltpu.VMEM((CH, H), jnp.float32),
            pltpu.VMEM((CH, H), jnp.float32),
            pltpu.SemaphoreType.DMA,
            pltpu.SemaphoreType.DMA,
            pltpu.SemaphoreType.DMA,
            pltpu.SemaphoreType.DMA,
        ])
    def sc_gather(q_hbm, k_hbm, v_hbm, dst_hbm, src_hbm,
                  qd_hbm, ks_hbm, vs_hbm,
                  idxd_v, idxs_v, bufQ, bufK, bufV, sem1, sem2, sem3, semw):
        cid = lax.axis_index("c")
        sid = lax.axis_index("s")
        wid = sid * NC + cid

        def body(g, carry):
            base = wid * PER_W + g * CH
            pltpu.sync_copy(dst_hbm.at[pl.ds(base, CH)], idxd_v)
            pltpu.sync_copy(src_hbm.at[pl.ds(base, CH)], idxs_v)
            d1 = pltpu.async_copy(q_hbm.at[idxd_v], bufQ, sem1)
            d2 = pltpu.async_copy(k_hbm.at[idxs_v], bufK, sem2)
            d3 = pltpu.async_copy(v_hbm.at[idxs_v], bufV, sem3)
            d1.wait()
            w1 = pltpu.async_copy(bufQ, qd_hbm.at[pl.ds(base, CH)], semw)
            d2.wait()
            w2 = pltpu.async_copy(bufK, ks_hbm.at[pl.ds(base, CH)], semw)
            d3.wait()
            w3 = pltpu.async_copy(bufV, vs_hbm.at[pl.ds(base, CH)], semw)
            w1.wait()
            w2.wait()
            w3.wait()
            return carry

        lax.fori_loop(0, NCHUNK, body, 0)

    _SC_SCATTER_CACHE["g"] = sc_gather
    return sc_gather


def _norm_proj_body(p0_ref, p1_ref, s0_ref, s1_ref, w_ref, b_ref, o_ref):
    uagg = p0_ref[...] + p1_ref[...]
    ssb = s0_ref[...] + s1_ref[...]
    ssb = jnp.where(ssb == 0.0, 1.0, ssb)
    agg = uagg / ssb
    o_ref[...] = jnp.dot(
        agg, w_ref[...], preferred_element_type=jnp.float32) + b_ref[...]


def _norm_proj(p128, s128, w, b):
    return pl.pallas_call(
        _norm_proj_body,
        grid=(NPAD // ROWS_BLK,),
        in_specs=[
            pl.BlockSpec((ROWS_BLK, H), lambda i: (i, 0)),
            pl.BlockSpec((ROWS_BLK, H), lambda i: (i, 0)),
            pl.BlockSpec((ROWS_BLK, H), lambda i: (i, 0)),
            pl.BlockSpec((ROWS_BLK, H), lambda i: (i, 0)),
            pl.BlockSpec((H, H), lambda i: (0, 0)),
            pl.BlockSpec((1, H), lambda i: (0, 0)),
        ],
        out_specs=pl.BlockSpec((ROWS_BLK, H), lambda i: (i, 0)),
        out_shape=jax.ShapeDtypeStruct((NPAD, H), jnp.float32),
    )(p128[0], p128[1], s128[0], s128[1], w, b.reshape(1, H))


def kernel(x, edge_index, batch, timestamps, W_in, b_in, time_freq,
           Wq, bq, Wk, bk, Wv, bv, Wo, bo, W1, b1, W2, b2):
    src = edge_index[0]
    dst = edge_index[1]
    pad_idx = jnp.full((EP - E,), NPAD - 1, jnp.int32)
    src_p = jnp.concatenate([src, pad_idx])
    dst_p = jnp.concatenate([dst, pad_idx])

    t_norm = (timestamps - jnp.min(timestamps)) / (
        jnp.max(timestamps) - jnp.min(timestamps) + 1e-08)
    om = t_norm[:, None] * time_freq[None, :]
    tf_e = jnp.concatenate(
        [jnp.cos(om), jnp.sin(om), jnp.ones((E, 1), jnp.float32),
         jnp.zeros((E, H - TD - 1), jnp.float32)], axis=1)
    msg_tf = jnp.concatenate(
        [tf_e, jnp.zeros((EP - E, H), jnp.float32)], axis=0)
    tf_part = _make_sc_scatter()(msg_tf, dst_p)
    tot = tf_part[0] + tf_part[1]
    node_tf_p = tot[:, :TD] / (tot[:, TD:TD + 1] + 1e-08)

    xp = jnp.zeros((NPAD, D_IN), jnp.float32).at[:N].set(x)
    h = _proj(xp, W_in, b_in)

    for l in range(NL):
        xt = jnp.concatenate([h, node_tf_p], axis=-1)
        Q = _proj(xt, Wq[l], bq[l])
        K = _proj(xt, Wk[l], bk[l])
        V = _proj(h, Wv[l], bv[l])
        Qd, Ks, Vs = _make_sc_gather3()(Q, K, V, dst_p, src_p)
        msg, exb = _edge_fused(Qd, Ks, Vs)
        p128 = _make_sc_scatter()(msg, dst_p)
        s128 = _make_sc_scatter()(exb, dst_p)
        out = _norm_proj((p128[0], p128[1]), (s128[0], s128[1]), Wo[l], bo[l])
        h = jax.nn.elu(out)

    hN = h[:N]
    g = jnp.mean(hN, axis=0, keepdims=True)
    logits = jax.nn.relu(g @ W1 + b1) @ W2 + b2
    return logits
